# baseline (speedup 1.0000x reference)
import functools

import jax
import jax.numpy as jnp
from jax import lax
from jax.experimental import pallas as pl
from jax.experimental.pallas import tpu as pltpu

T = 512
A = 16
NE = 22
RAW = 128
H = 64
NA = 32
NF_F = 2
NF_H = 3

BT = 16

_WDEFS = (
    ("W1", ("W1",)), ("b1", ("b1",)), ("W2", ("W2",)), ("b2", ("b2",)),
    ("fWq", ("conc_f", "Wq")), ("fWk", ("conc_f", "Wk")),
    ("fWsk", ("conc_f", "Wsk")), ("fbsk", ("conc_f", "bsk")),
    ("fWc", ("conc_f", "Wc")), ("fbc", ("conc_f", "bc")),
    ("fWm", ("conc_f", "Wm")), ("fbm", ("conc_f", "bm")),
    ("hWq", ("conc_h", "Wq")), ("hWk", ("conc_h", "Wk")),
    ("hWsk", ("conc_h", "Wsk")), ("hbsk", ("conc_h", "bsk")),
    ("hWc", ("conc_h", "Wc")), ("hbc", ("conc_h", "bc")),
    ("hWm", ("conc_h", "Wm")), ("hbm", ("conc_h", "bm")),
    ("Wl1", ("Wl1",)), ("bl1", ("bl1",)), ("Wl2", ("Wl2",)),
    ("bl2", ("bl2",)), ("Wl3", ("Wl3",)), ("bl3", ("bl3",)),
    ("Wv1", ("Wv1",)), ("bv1", ("bv1",)), ("Wv2", ("Wv2",)),
    ("bv2", ("bv2",)),
)


def _relu(x):
    return jnp.maximum(x, 0.0)


def _first_argmax(x):
    n = x.shape[-1]
    iot = lax.broadcasted_iota(jnp.int32, x.shape, x.ndim - 1)
    m = jnp.max(x, axis=-1, keepdims=True)
    idx = jnp.min(jnp.where(x == m, iot, n), axis=-1, keepdims=True)
    return (iot == idx), m


def _load_tok(obs_ref, e, ent, r):
    z = obs_ref[:, e * 8:(e + 1) * 8, ent, :].reshape(r, RAW)
    nan = z != z
    dead = jnp.max(jnp.where(nan, 1.0, 0.0), axis=-1, keepdims=True) > 0.0
    return jnp.where(nan, 0.0, z), dead


def _concentrate(r, vs_f, v_by_ent, s_by_ent, dead_by_ent, lo, n, k, w, p):
    q = vs_f @ w[p + "Wq"]
    cols = []
    for j in range(n):
        kk = v_by_ent[lo + j] @ w[p + "Wk"]
        sc = jnp.sum(q * kk, axis=-1, keepdims=True) * 0.125
        cols.append(jnp.where(dead_by_ent[lo + j], -1e9, sc))
    score = jnp.concatenate(cols, axis=-1)

    merged = []
    for rank in range(k):
        oh, _ = _first_argmax(score)
        ohf = oh.astype(jnp.float32)
        sel_s = jnp.zeros((r, H), jnp.float32)
        for j in range(n):
            sel_s = sel_s + ohf[:, j:j + 1] * s_by_ent[lo + j]
        merged.append(_relu(sel_s + w[p + "bsk"]))
        if rank < k - 1:
            score = jnp.where(oh, -3e38, score)

    c_in = jnp.concatenate([vs_f] + merged, axis=-1)
    big_c = _relu(c_in @ w[p + "Wc"] + w[p + "bc"])
    mx = merged[0]
    for m in merged[1:]:
        mx = jnp.maximum(mx, m)
    m_in = jnp.concatenate([vs_f, mx], axis=-1)
    big_m = _relu(m_in @ w[p + "Wm"] + w[p + "bm"])
    return big_c, big_m


def _body(*refs):
    nw = len(_WDEFS)
    obs_ref = refs[0]
    wrefs = refs[1:1 + 2 * nw]
    act_ref, val_ref, alp_ref = refs[1 + 2 * nw:]
    r = BT * 8

    for e in range(2):
        w = {name: wref[...]
             for (name, _), wref in zip(_WDEFS, wrefs[e * nw:(e + 1) * nw])}

        v_by_ent = []
        s_by_ent = []
        dead_by_ent = []
        for ent in range(NE):
            z, dead = _load_tok(obs_ref, e, ent, r)
            v = _relu(z @ w["W1"] + w["b1"]) @ w["W2"] + w["b2"]
            if 1 <= ent < 12:
                s = v @ w["fWsk"][:H] + z @ w["fWsk"][H:]
            elif ent >= 12:
                s = v @ w["hWsk"][:H] + z @ w["hWsk"][H:]
            else:
                s = None
            v_by_ent.append(v)
            s_by_ent.append(s)
            dead_by_ent.append(dead)
        vs_f = v_by_ent[0]

        f_c, f_m = _concentrate(r, vs_f, v_by_ent, s_by_ent, dead_by_ent,
                                1, 11, NF_F, w, "f")
        h_c, h_m = _concentrate(r, vs_f, v_by_ent, s_by_ent, dead_by_ent,
                                12, 10, NF_H, w, "h")

        v_c = jnp.concatenate([f_c, h_c], axis=-1)
        v_m = jnp.concatenate([f_m, h_m], axis=-1)
        h1 = _relu(v_c @ w["Wl1"] + w["bl1"])
        h2 = _relu(h1 @ w["Wl2"] + w["bl2"])
        logits = h2 @ w["Wl3"] + w["bl3"]
        value = _relu(v_m @ w["Wv1"] + w["bv1"]) @ w["Wv2"] + w["bv2"]

        mx = jnp.max(logits, axis=-1, keepdims=True)
        shifted = logits - mx
        lse = jnp.log(jnp.sum(jnp.exp(shifted), axis=-1, keepdims=True))
        logp = shifted - lse
        oh, _ = _first_argmax(logp)
        iot = lax.broadcasted_iota(jnp.int32, logits.shape, 1)
        act = jnp.min(jnp.where(oh, iot, NA), axis=-1, keepdims=True)
        alp = -lse

        cols = slice(e * 8, (e + 1) * 8)
        act_ref[:, cols] = act.reshape(BT, 8)
        val_ref[:, cols] = value.reshape(BT, 8)
        alp_ref[:, cols] = alp.reshape(BT, 8)


@jax.jit
def _run(obs, weights):
    grid = (T // BT,)
    wspecs = [
        pl.BlockSpec(wa.shape, lambda i, _nd=wa.ndim: (0,) * _nd)
        for wa in weights
    ]
    out = pl.pallas_call(
        _body,
        grid=grid,
        in_specs=[pl.BlockSpec((BT, A, NE, RAW), lambda i: (i, 0, 0, 0))]
        + wspecs,
        out_specs=[
            pl.BlockSpec((BT, A), lambda i: (i, 0)),
            pl.BlockSpec((BT, A), lambda i: (i, 0)),
            pl.BlockSpec((BT, A), lambda i: (i, 0)),
        ],
        out_shape=[
            jax.ShapeDtypeStruct((T, A), jnp.int32),
            jax.ShapeDtypeStruct((T, A), jnp.float32),
            jax.ShapeDtypeStruct((T, A), jnp.float32),
        ],
    )(obs, *weights)
    act, value, alp = out
    return act, value.reshape(T, A, 1), alp.reshape(T, A, 1)


def kernel(obs, params, test_mode):
    ex = params["experts"]

    def get(p, path):
        o = p
        for kk in path:
            o = o[kk]
        return o

    weights = []
    for e in range(2):
        for _, path in _WDEFS:
            a = get(ex[e], path)
            if a.ndim == 1:
                a = a.reshape(1, -1)
            weights.append(a)
    return _run(obs, weights)


# speedup vs baseline: 1.0964x; 1.0964x over previous
import functools

import jax
import jax.numpy as jnp
from jax import lax
from jax.experimental import pallas as pl
from jax.experimental.pallas import tpu as pltpu

T = 512
A = 16
NE = 22
RAW = 128
H = 64
NA = 32
NF_F = 2
NF_H = 3

BT = 16

_WDEFS = (
    ("W1", ("W1",)), ("b1", ("b1",)), ("W2", ("W2",)), ("b2", ("b2",)),
    ("fWq", ("conc_f", "Wq")), ("fWk", ("conc_f", "Wk")),
    ("fWsk", ("conc_f", "Wsk")), ("fbsk", ("conc_f", "bsk")),
    ("fWc", ("conc_f", "Wc")), ("fbc", ("conc_f", "bc")),
    ("fWm", ("conc_f", "Wm")), ("fbm", ("conc_f", "bm")),
    ("hWq", ("conc_h", "Wq")), ("hWk", ("conc_h", "Wk")),
    ("hWsk", ("conc_h", "Wsk")), ("hbsk", ("conc_h", "bsk")),
    ("hWc", ("conc_h", "Wc")), ("hbc", ("conc_h", "bc")),
    ("hWm", ("conc_h", "Wm")), ("hbm", ("conc_h", "bm")),
    ("Wl1", ("Wl1",)), ("bl1", ("bl1",)), ("Wl2", ("Wl2",)),
    ("bl2", ("bl2",)), ("Wl3", ("Wl3",)), ("bl3", ("bl3",)),
    ("Wv1", ("Wv1",)), ("bv1", ("bv1",)), ("Wv2", ("Wv2",)),
    ("bv2", ("bv2",)),
)


def _relu(x):
    return jnp.maximum(x, 0.0)


def _first_argmax(x):
    n = x.shape[-1]
    iot = lax.broadcasted_iota(jnp.int32, x.shape, x.ndim - 1)
    m = jnp.max(x, axis=-1, keepdims=True)
    idx = jnp.min(jnp.where(x == m, iot, n), axis=-1, keepdims=True)
    return (iot == idx), m


def _load_tok(obs_ref, e, ent, r):
    z = obs_ref[:, e * 8:(e + 1) * 8, ent, :].reshape(r, RAW)
    nan = z != z
    dead = jnp.max(jnp.where(nan, 1.0, 0.0), axis=-1, keepdims=True) > 0.0
    return jnp.where(nan, 0.0, z), dead


def _concentrate(r, vs_f, v_by_ent, zs_ref, dead_by_ent, lo, n, k, w, p):
    q = vs_f @ w[p + "Wq"]
    cols = []
    for j in range(n):
        kk = v_by_ent[lo + j] @ w[p + "Wk"]
        sc = jnp.sum(q * kk, axis=-1, keepdims=True) * 0.125
        cols.append(jnp.where(dead_by_ent[lo + j], -1e9, sc))
    score = jnp.concatenate(cols, axis=-1)

    merged = []
    for rank in range(k):
        oh, _ = _first_argmax(score)
        ohf = oh.astype(jnp.float32)
        sel_v = jnp.zeros((r, H), jnp.float32)
        sel_z = jnp.zeros((r, RAW), jnp.float32)
        for j in range(n):
            col = ohf[:, j:j + 1]
            sel_v = sel_v + col * v_by_ent[lo + j]
            sel_z = sel_z + col * zs_ref[lo + j]
        merged.append(_relu(
            jnp.concatenate([sel_v, sel_z], axis=-1) @ w[p + "Wsk"]
            + w[p + "bsk"]))
        if rank < k - 1:
            score = jnp.where(oh, -3e38, score)

    c_in = jnp.concatenate([vs_f] + merged, axis=-1)
    big_c = _relu(c_in @ w[p + "Wc"] + w[p + "bc"])
    mx = merged[0]
    for m in merged[1:]:
        mx = jnp.maximum(mx, m)
    m_in = jnp.concatenate([vs_f, mx], axis=-1)
    big_m = _relu(m_in @ w[p + "Wm"] + w[p + "bm"])
    return big_c, big_m


def _body(*refs):
    nw = len(_WDEFS)
    obs_ref = refs[0]
    wrefs = refs[1:1 + 2 * nw]
    act_ref, val_ref, alp_ref = refs[1 + 2 * nw:-1]
    zs_ref = refs[-1]
    r = BT * 8

    for e in range(2):
        w = {name: wref[...]
             for (name, _), wref in zip(_WDEFS, wrefs[e * nw:(e + 1) * nw])}

        v_by_ent = []
        dead_by_ent = []
        for ent in range(NE):
            z, dead = _load_tok(obs_ref, e, ent, r)
            zs_ref[ent] = z
            v = _relu(z @ w["W1"] + w["b1"]) @ w["W2"] + w["b2"]
            v_by_ent.append(v)
            dead_by_ent.append(dead)
        vs_f = v_by_ent[0]

        f_c, f_m = _concentrate(r, vs_f, v_by_ent, zs_ref, dead_by_ent,
                                1, 11, NF_F, w, "f")
        h_c, h_m = _concentrate(r, vs_f, v_by_ent, zs_ref, dead_by_ent,
                                12, 10, NF_H, w, "h")

        v_c = jnp.concatenate([f_c, h_c], axis=-1)
        v_m = jnp.concatenate([f_m, h_m], axis=-1)
        h1 = _relu(v_c @ w["Wl1"] + w["bl1"])
        h2 = _relu(h1 @ w["Wl2"] + w["bl2"])
        logits = h2 @ w["Wl3"] + w["bl3"]
        value = _relu(v_m @ w["Wv1"] + w["bv1"]) @ w["Wv2"] + w["bv2"]

        mx = jnp.max(logits, axis=-1, keepdims=True)
        shifted = logits - mx
        lse = jnp.log(jnp.sum(jnp.exp(shifted), axis=-1, keepdims=True))
        logp = shifted - lse
        oh, _ = _first_argmax(logp)
        iot = lax.broadcasted_iota(jnp.int32, logits.shape, 1)
        act = jnp.min(jnp.where(oh, iot, NA), axis=-1, keepdims=True)
        alp = -lse

        cols = slice(e * 8, (e + 1) * 8)
        act_ref[:, cols] = act.reshape(BT, 8)
        val_ref[:, cols] = value.reshape(BT, 8)
        alp_ref[:, cols] = alp.reshape(BT, 8)


@jax.jit
def _run(obs, weights):
    grid = (T // BT,)
    wspecs = [
        pl.BlockSpec(wa.shape, lambda i, _nd=wa.ndim: (0,) * _nd)
        for wa in weights
    ]
    out = pl.pallas_call(
        _body,
        grid=grid,
        in_specs=[pl.BlockSpec((BT, A, NE, RAW), lambda i: (i, 0, 0, 0))]
        + wspecs,
        out_specs=[
            pl.BlockSpec((BT, A), lambda i: (i, 0)),
            pl.BlockSpec((BT, A), lambda i: (i, 0)),
            pl.BlockSpec((BT, A), lambda i: (i, 0)),
        ],
        out_shape=[
            jax.ShapeDtypeStruct((T, A), jnp.int32),
            jax.ShapeDtypeStruct((T, A), jnp.float32),
            jax.ShapeDtypeStruct((T, A), jnp.float32),
        ],
        scratch_shapes=[pltpu.VMEM((NE, BT * 8, RAW), jnp.float32)],
    )(obs, *weights)
    act, value, alp = out
    return act, value.reshape(T, A, 1), alp.reshape(T, A, 1)


def kernel(obs, params, test_mode):
    ex = params["experts"]

    def get(p, path):
        o = p
        for kk in path:
            o = o[kk]
        return o

    weights = []
    for e in range(2):
        for _, path in _WDEFS:
            a = get(ex[e], path)
            if a.ndim == 1:
                a = a.reshape(1, -1)
            weights.append(a)
    return _run(obs, weights)


# speedup vs baseline: 1.0966x; 1.0002x over previous
import functools

import jax
import jax.numpy as jnp
from jax import lax
from jax.experimental import pallas as pl
from jax.experimental.pallas import tpu as pltpu

T = 512
A = 16
NE = 22
RAW = 128
H = 64
NA = 32
NF_F = 2
NF_H = 3

BT = 16

_WDEFS = (
    ("W1", ("W1",)), ("b1", ("b1",)), ("W2", ("W2",)), ("b2", ("b2",)),
    ("fWq", ("conc_f", "Wq")), ("fWk", ("conc_f", "Wk")),
    ("fWsk", ("conc_f", "Wsk")), ("fbsk", ("conc_f", "bsk")),
    ("fWc", ("conc_f", "Wc")), ("fbc", ("conc_f", "bc")),
    ("fWm", ("conc_f", "Wm")), ("fbm", ("conc_f", "bm")),
    ("hWq", ("conc_h", "Wq")), ("hWk", ("conc_h", "Wk")),
    ("hWsk", ("conc_h", "Wsk")), ("hbsk", ("conc_h", "bsk")),
    ("hWc", ("conc_h", "Wc")), ("hbc", ("conc_h", "bc")),
    ("hWm", ("conc_h", "Wm")), ("hbm", ("conc_h", "bm")),
    ("Wl1", ("Wl1",)), ("bl1", ("bl1",)), ("Wl2", ("Wl2",)),
    ("bl2", ("bl2",)), ("Wl3", ("Wl3",)), ("bl3", ("bl3",)),
    ("Wv1", ("Wv1",)), ("bv1", ("bv1",)), ("Wv2", ("Wv2",)),
    ("bv2", ("bv2",)),
)


def _relu(x):
    return jnp.maximum(x, 0.0)


def _first_argmax(x):
    n = x.shape[-1]
    iot = lax.broadcasted_iota(jnp.int32, x.shape, x.ndim - 1)
    m = jnp.max(x, axis=-1, keepdims=True)
    idx = jnp.min(jnp.where(x == m, iot, n), axis=-1, keepdims=True)
    return (iot == idx), m


def _load_tok(obs_ref, e, ent, r):
    z = obs_ref[:, e * 8:(e + 1) * 8, ent, :].reshape(r, RAW)
    nan = z != z
    dead = jnp.max(jnp.where(nan, 1.0, 0.0), axis=-1, keepdims=True) > 0.0
    return jnp.where(nan, 0.0, z), dead


def _concentrate(r, vs_f, v_by_ent, zs_ref, dead_by_ent, lo, n, k, w, p):
    q = vs_f @ w[p + "Wq"]
    cols = []
    for j in range(n):
        kk = v_by_ent[lo + j] @ w[p + "Wk"]
        sc = jnp.sum(q * kk, axis=-1, keepdims=True) * 0.125
        cols.append(jnp.where(dead_by_ent[lo + j], -1e9, sc))
    score = jnp.concatenate(cols, axis=-1)

    merged = []
    for rank in range(k):
        oh, _ = _first_argmax(score)
        ohf = oh.astype(jnp.float32)
        sel_v = jnp.zeros((r, H), jnp.float32)
        sel_z = jnp.zeros((r, RAW), jnp.float32)
        for j in range(n):
            col = ohf[:, j:j + 1]
            sel_v = sel_v + col * v_by_ent[lo + j]
            sel_z = sel_z + col * zs_ref[lo + j]
        merged.append(_relu(
            jnp.concatenate([sel_v, sel_z], axis=-1) @ w[p + "Wsk"]
            + w[p + "bsk"]))
        if rank < k - 1:
            score = jnp.where(oh, -3e38, score)

    c_in = jnp.concatenate([vs_f] + merged, axis=-1)
    big_c = _relu(c_in @ w[p + "Wc"] + w[p + "bc"])
    mx = merged[0]
    for m in merged[1:]:
        mx = jnp.maximum(mx, m)
    m_in = jnp.concatenate([vs_f, mx], axis=-1)
    big_m = _relu(m_in @ w[p + "Wm"] + w[p + "bm"])
    return big_c, big_m


def _body(*refs):
    nw = len(_WDEFS)
    obs_ref = refs[0]
    wrefs = refs[1:1 + 2 * nw]
    act_ref, val_ref, alp_ref = refs[1 + 2 * nw:-1]
    zs_ref = refs[-1]
    r = BT * 8

    for e in range(2):
        w = {name: wref[...]
             for (name, _), wref in zip(_WDEFS, wrefs[e * nw:(e + 1) * nw])}

        v_by_ent = []
        dead_by_ent = []
        for ent in range(NE):
            z, dead = _load_tok(obs_ref, e, ent, r)
            zs_ref[ent] = z
            v = _relu(z @ w["W1"] + w["b1"]) @ w["W2"] + w["b2"]
            v_by_ent.append(v)
            dead_by_ent.append(dead)
        vs_f = v_by_ent[0]

        f_c, f_m = _concentrate(r, vs_f, v_by_ent, zs_ref, dead_by_ent,
                                1, 11, NF_F, w, "f")
        h_c, h_m = _concentrate(r, vs_f, v_by_ent, zs_ref, dead_by_ent,
                                12, 10, NF_H, w, "h")

        v_c = jnp.concatenate([f_c, h_c], axis=-1)
        v_m = jnp.concatenate([f_m, h_m], axis=-1)
        h1 = _relu(v_c @ w["Wl1"] + w["bl1"])
        h2 = _relu(h1 @ w["Wl2"] + w["bl2"])
        logits = h2 @ w["Wl3"] + w["bl3"]
        value = _relu(v_m @ w["Wv1"] + w["bv1"]) @ w["Wv2"] + w["bv2"]

        mx = jnp.max(logits, axis=-1, keepdims=True)
        shifted = logits - mx
        lse = jnp.log(jnp.sum(jnp.exp(shifted), axis=-1, keepdims=True))
        logp = shifted - lse
        oh, _ = _first_argmax(logp)
        iot = lax.broadcasted_iota(jnp.int32, logits.shape, 1)
        act = jnp.min(jnp.where(oh, iot, NA), axis=-1, keepdims=True)
        alp = -lse

        cols = slice(e * 8, (e + 1) * 8)
        act_ref[:, cols] = act.reshape(BT, 8)
        val_ref[:, cols] = value.reshape(BT, 8)
        alp_ref[:, cols] = alp.reshape(BT, 8)


@jax.jit
def _run(obs, weights):
    grid = (T // BT,)
    wspecs = [
        pl.BlockSpec(wa.shape, lambda i, _nd=wa.ndim: (0,) * _nd)
        for wa in weights
    ]
    out = pl.pallas_call(
        _body,
        grid=grid,
        in_specs=[pl.BlockSpec((BT, A, NE, RAW), lambda i: (i, 0, 0, 0))]
        + wspecs,
        out_specs=[
            pl.BlockSpec((BT, A), lambda i: (i, 0)),
            pl.BlockSpec((BT, A), lambda i: (i, 0)),
            pl.BlockSpec((BT, A), lambda i: (i, 0)),
        ],
        out_shape=[
            jax.ShapeDtypeStruct((T, A), jnp.int32),
            jax.ShapeDtypeStruct((T, A), jnp.float32),
            jax.ShapeDtypeStruct((T, A), jnp.float32),
        ],
        scratch_shapes=[pltpu.VMEM((NE, BT * 8, RAW), jnp.float32)],
        compiler_params=pltpu.CompilerParams(
            dimension_semantics=("parallel",)),
    )(obs, *weights)
    act, value, alp = out
    return act, value.reshape(T, A, 1), alp.reshape(T, A, 1)


def kernel(obs, params, test_mode):
    ex = params["experts"]

    def get(p, path):
        o = p
        for kk in path:
            o = o[kk]
        return o

    weights = []
    for e in range(2):
        for _, path in _WDEFS:
            a = get(ex[e], path)
            if a.ndim == 1:
                a = a.reshape(1, -1)
            weights.append(a)
    return _run(obs, weights)


# speedup vs baseline: 1.2388x; 1.1297x over previous
import functools

import jax
import jax.numpy as jnp
from jax import lax
from jax.experimental import pallas as pl
from jax.experimental.pallas import tpu as pltpu

T = 512
A = 16
NE = 22
RAW = 128
H = 64
NA = 32
NF_F = 2
NF_H = 3

BT = 32

_WDEFS = (
    ("W1", ("W1",)), ("b1", ("b1",)), ("W2", ("W2",)), ("b2", ("b2",)),
    ("fWq", ("conc_f", "Wq")), ("fWk", ("conc_f", "Wk")),
    ("fWsk", ("conc_f", "Wsk")), ("fbsk", ("conc_f", "bsk")),
    ("fWc", ("conc_f", "Wc")), ("fbc", ("conc_f", "bc")),
    ("fWm", ("conc_f", "Wm")), ("fbm", ("conc_f", "bm")),
    ("hWq", ("conc_h", "Wq")), ("hWk", ("conc_h", "Wk")),
    ("hWsk", ("conc_h", "Wsk")), ("hbsk", ("conc_h", "bsk")),
    ("hWc", ("conc_h", "Wc")), ("hbc", ("conc_h", "bc")),
    ("hWm", ("conc_h", "Wm")), ("hbm", ("conc_h", "bm")),
    ("Wl1", ("Wl1",)), ("bl1", ("bl1",)), ("Wl2", ("Wl2",)),
    ("bl2", ("bl2",)), ("Wl3", ("Wl3",)), ("bl3", ("bl3",)),
    ("Wv1", ("Wv1",)), ("bv1", ("bv1",)), ("Wv2", ("Wv2",)),
    ("bv2", ("bv2",)),
)


def _relu(x):
    return jnp.maximum(x, 0.0)


def _first_argmax(x):
    n = x.shape[-1]
    iot = lax.broadcasted_iota(jnp.int32, x.shape, x.ndim - 1)
    m = jnp.max(x, axis=-1, keepdims=True)
    idx = jnp.min(jnp.where(x == m, iot, n), axis=-1, keepdims=True)
    return (iot == idx), m


def _load_tok(obs_ref, e, ent, r):
    z = obs_ref[:, e * 8:(e + 1) * 8, ent, :].reshape(r, RAW)
    nan = z != z
    dead = jnp.max(jnp.where(nan, 1.0, 0.0), axis=-1, keepdims=True) > 0.0
    return jnp.where(nan, 0.0, z), dead


def _concentrate(r, vs_f, v_by_ent, zs_ref, dead_by_ent, lo, n, k, w, p):
    q = vs_f @ w[p + "Wq"]
    cols = []
    for j in range(n):
        kk = v_by_ent[lo + j] @ w[p + "Wk"]
        sc = jnp.sum(q * kk, axis=-1, keepdims=True) * 0.125
        cols.append(jnp.where(dead_by_ent[lo + j], -1e9, sc))
    score = jnp.concatenate(cols, axis=-1)

    merged = []
    for rank in range(k):
        oh, _ = _first_argmax(score)
        ohf = oh.astype(jnp.float32)
        sel_v = jnp.zeros((r, H), jnp.float32)
        sel_z = jnp.zeros((r, RAW), jnp.float32)
        for j in range(n):
            col = ohf[:, j:j + 1]
            sel_v = sel_v + col * v_by_ent[lo + j]
            sel_z = sel_z + col * zs_ref[lo + j]
        merged.append(_relu(
            jnp.concatenate([sel_v, sel_z], axis=-1) @ w[p + "Wsk"]
            + w[p + "bsk"]))
        if rank < k - 1:
            score = jnp.where(oh, -3e38, score)

    c_in = jnp.concatenate([vs_f] + merged, axis=-1)
    big_c = _relu(c_in @ w[p + "Wc"] + w[p + "bc"])
    mx = merged[0]
    for m in merged[1:]:
        mx = jnp.maximum(mx, m)
    m_in = jnp.concatenate([vs_f, mx], axis=-1)
    big_m = _relu(m_in @ w[p + "Wm"] + w[p + "bm"])
    return big_c, big_m


def _body(*refs):
    nw = len(_WDEFS)
    obs_ref = refs[0]
    wrefs = refs[1:1 + 2 * nw]
    act_ref, val_ref, alp_ref = refs[1 + 2 * nw:-1]
    zs_ref = refs[-1]
    r = BT * 8

    for e in range(2):
        w = {name: wref[...]
             for (name, _), wref in zip(_WDEFS, wrefs[e * nw:(e + 1) * nw])}

        v_by_ent = []
        dead_by_ent = []
        for ent in range(NE):
            z, dead = _load_tok(obs_ref, e, ent, r)
            zs_ref[ent] = z
            v = _relu(z @ w["W1"] + w["b1"]) @ w["W2"] + w["b2"]
            v_by_ent.append(v)
            dead_by_ent.append(dead)
        vs_f = v_by_ent[0]

        f_c, f_m = _concentrate(r, vs_f, v_by_ent, zs_ref, dead_by_ent,
                                1, 11, NF_F, w, "f")
        h_c, h_m = _concentrate(r, vs_f, v_by_ent, zs_ref, dead_by_ent,
                                12, 10, NF_H, w, "h")

        v_c = jnp.concatenate([f_c, h_c], axis=-1)
        v_m = jnp.concatenate([f_m, h_m], axis=-1)
        h1 = _relu(v_c @ w["Wl1"] + w["bl1"])
        h2 = _relu(h1 @ w["Wl2"] + w["bl2"])
        logits = h2 @ w["Wl3"] + w["bl3"]
        value = _relu(v_m @ w["Wv1"] + w["bv1"]) @ w["Wv2"] + w["bv2"]

        mx = jnp.max(logits, axis=-1, keepdims=True)
        shifted = logits - mx
        lse = jnp.log(jnp.sum(jnp.exp(shifted), axis=-1, keepdims=True))
        logp = shifted - lse
        oh, _ = _first_argmax(logp)
        iot = lax.broadcasted_iota(jnp.int32, logits.shape, 1)
        act = jnp.min(jnp.where(oh, iot, NA), axis=-1, keepdims=True)
        alp = -lse

        cols = slice(e * 8, (e + 1) * 8)
        act_ref[:, cols] = act.reshape(BT, 8)
        val_ref[:, cols] = value.reshape(BT, 8)
        alp_ref[:, cols] = alp.reshape(BT, 8)


@jax.jit
def _run(obs, weights):
    grid = (T // BT,)
    wspecs = [
        pl.BlockSpec(wa.shape, lambda i, _nd=wa.ndim: (0,) * _nd)
        for wa in weights
    ]
    out = pl.pallas_call(
        _body,
        grid=grid,
        in_specs=[pl.BlockSpec((BT, A, NE, RAW), lambda i: (i, 0, 0, 0))]
        + wspecs,
        out_specs=[
            pl.BlockSpec((BT, A), lambda i: (i, 0)),
            pl.BlockSpec((BT, A), lambda i: (i, 0)),
            pl.BlockSpec((BT, A), lambda i: (i, 0)),
        ],
        out_shape=[
            jax.ShapeDtypeStruct((T, A), jnp.int32),
            jax.ShapeDtypeStruct((T, A), jnp.float32),
            jax.ShapeDtypeStruct((T, A), jnp.float32),
        ],
        scratch_shapes=[pltpu.VMEM((NE, BT * 8, RAW), jnp.float32)],
        compiler_params=pltpu.CompilerParams(
            dimension_semantics=("parallel",)),
    )(obs, *weights)
    act, value, alp = out
    return act, value.reshape(T, A, 1), alp.reshape(T, A, 1)


def kernel(obs, params, test_mode):
    ex = params["experts"]

    def get(p, path):
        o = p
        for kk in path:
            o = o[kk]
        return o

    weights = []
    for e in range(2):
        for _, path in _WDEFS:
            a = get(ex[e], path)
            if a.ndim == 1:
                a = a.reshape(1, -1)
            weights.append(a)
    return _run(obs, weights)
